# baseline (speedup 1.0000x reference)
import jax
import jax.numpy as jnp
from jax import lax
from jax.experimental import pallas as pl
from jax.experimental.pallas import tpu as pltpu
from jax.experimental.pallas import tpu_sc as plsc

_B, _C, _L = 4, 768, 8192
_H = 2048
_KEEP = 64
_LB1 = 1024
_CH2 = 2048
_LB3 = 1024
_NEG = -3.0e38
_PREC = lax.Precision.HIGHEST
_ZR = 8


def _p1_body(x_ref, w1_ref, b1_ref, out_ref):
    hb = lax.dot_general(w1_ref[...], x_ref[0], (((1,), (0,)), ((), ())),
                         preferred_element_type=jnp.float32)
    hb = hb + b1_ref[...]
    out_ref[0, 0, :] = jnp.max(hb, axis=0)


def _p2_body(cm_ref, x_ref, w1_ref, b1_ref,
             v_ref, hh_ref, ll_ref, spt_ref, qi_ref,
             hgT_ref, xgT_ref, cols_ref):
    iota_l = lax.broadcasted_iota(jnp.int32, (1, _L), 1)
    iota_kr = lax.broadcasted_iota(jnp.int32, (1, _KEEP), 1)
    iota_kc = lax.broadcasted_iota(jnp.int32, (_KEEP, 1), 0)
    iota_h = lax.broadcasted_iota(jnp.int32, (1, _H), 1)
    c = pl.program_id(1)

    @pl.when(c == 0)
    def _():
        def sel_body(k, carry):
            cm, cols = carry
            m = jnp.max(cm)
            j = jnp.min(jnp.where(cm == m, iota_l, _L))
            cm = jnp.where(iota_l == j, _NEG, cm)
            cols = jnp.where(iota_kc == k, j, cols)
            return cm, cols

        cm0 = cm_ref[0]
        cols0 = jnp.zeros((_KEEP, 1), jnp.int32)
        _, cols = lax.fori_loop(0, _KEEP, sel_body, (cm0, cols0))
        cols_ref[...] = cols
        xgT_ref[...] = jnp.zeros((_KEEP, _C), jnp.float32)

    cols = cols_ref[...]
    selm = (c * _CH2 + lax.broadcasted_iota(jnp.int32, (_KEEP, _CH2), 1)
            == cols).astype(jnp.bfloat16)
    xgT_ref[...] += lax.dot_general(selm, x_ref[0], (((1,), (1,)), ((), ())),
                                    preferred_element_type=jnp.float32)

    @pl.when(c == _L // _CH2 - 1)
    def _():
        hgT = lax.dot_general(xgT_ref[...].astype(jnp.bfloat16), w1_ref[...],
                              (((1,), (1,)), ((), ())),
                              preferred_element_type=jnp.float32)
        hgT = hgT + b1_ref[...]
        hgT_ref[...] = hgT
        rmax0 = jnp.max(hgT, axis=1, keepdims=True)

        def ext_body(k, carry):
            rmax, v, hh, ll, hhc, vc, llc = carry
            m = jnp.max(rmax)
            j = jnp.min(jnp.where(rmax == m, iota_kc, _KEEP))
            row = hgT_ref[pl.ds(j, 1), :]
            i = jnp.min(jnp.where(row == m, iota_h, _H))
            masked = jnp.where(iota_h == i, _NEG, row)
            hgT_ref[pl.ds(j, 1), :] = masked
            nm = jnp.max(masked)
            rmax = jnp.where(iota_kc == j, nm, rmax)
            lcol = jnp.min(jnp.where(iota_kc == j, cols, _L))
            v = jnp.where(iota_kr == k, m, v)
            hh = jnp.where(iota_kr == k, i, hh)
            ll = jnp.where(iota_kr == k, lcol, ll)
            hhc = jnp.where(iota_kc == k, i, hhc)
            vc = jnp.where(iota_kc == k, m, vc)
            llc = jnp.where(iota_kc == k, lcol, llc)
            return rmax, v, hh, ll, hhc, vc, llc

        z0r = jnp.zeros((1, _KEEP), jnp.int32)
        v0r = jnp.zeros((1, _KEEP), jnp.float32)
        z0c = jnp.zeros((_KEEP, 1), jnp.int32)
        v0c = jnp.zeros((_KEEP, 1), jnp.float32)
        _, v, hh, ll, hhc, vc, llc = lax.fori_loop(
            0, _KEEP, ext_body,
            (rmax0, v0r, z0r, z0r, z0c, v0c, z0c))

        ld128r = ll // 128
        ld128c = llc // 128
        same_row = ((hhc == hh) & (ld128c == ld128r)).astype(jnp.float32)
        p128v = (lax.broadcasted_iota(jnp.int32, (_KEEP, 128), 1)
                 == (llc % 128)).astype(jnp.float32) * vc
        rows = lax.dot_general(same_row, p128v, (((1,), (0,)), ((), ())),
                               preferred_element_type=jnp.float32,
                               precision=_PREC)
        b = pl.program_id(0)
        v_ref[0] = v
        hh_ref[0] = hh
        ll_ref[0] = ll
        spt_ref[0] = rows
        qi_ref[0] = (b * _H + hh) * 64 + ld128r


def _sc_mesh():
    return plsc.VectorSubcoreMesh(core_axis_name="c", subcore_axis_name="s")


def _fill_body(spz_ref, zbuf):
    wid = lax.axis_index("s") * 2 + lax.axis_index("c")
    z16 = jnp.zeros((16,), jnp.float32)

    def zb(t, carry):
        r = t // (_L // 16)
        i = t % (_L // 16)
        zbuf[r, pl.ds(i * 16, 16)] = z16
        return carry

    lax.fori_loop(0, _ZR * (_L // 16), zb, 0)
    b = wid // 8
    sub = wid % 8
    for j in range(256 // _ZR):
        h0 = sub * 256 + j * _ZR
        pltpu.sync_copy(zbuf, spz_ref.at[b, pl.ds(h0, _ZR), :])


def _scat_body(spt_hbm, qi_hbm, spz2_ref, rbuf, idxv, sem):
    wid = lax.axis_index("s") * 2 + lax.axis_index("c")
    p0 = wid * 8
    pltpu.sync_copy(spt_hbm.at[pl.ds(p0, 8)], rbuf)
    pltpu.sync_copy(qi_hbm.at[pl.ds(p0, 8)], idxv)
    pltpu.async_copy(rbuf, spz2_ref.at[idxv], sem).wait()


def _tc_front(x, W1, b1, W2, b2):
    xb = x.astype(jnp.bfloat16)
    W1b = W1.astype(jnp.bfloat16)
    W2b = W2.astype(jnp.bfloat16)
    b1c = b1.reshape(_H, 1)
    b1r = b1.reshape(1, _H)
    b2r = b2.reshape(1, _C)

    colmax = pl.pallas_call(
        _p1_body,
        grid=(_B, _L // _LB1),
        in_specs=[
            pl.BlockSpec((1, _C, _LB1), lambda b, l: (b, 0, l)),
            pl.BlockSpec((_H, _C), lambda b, l: (0, 0)),
            pl.BlockSpec((_H, 1), lambda b, l: (0, 0)),
        ],
        out_specs=pl.BlockSpec((1, 1, _LB1), lambda b, l: (b, 0, l)),
        out_shape=jax.ShapeDtypeStruct((_B, 1, _L), jnp.float32),
    )(xb, W1b, b1c)

    v, hh, ll, spt, qi = pl.pallas_call(
        _p2_body,
        grid=(_B, _L // _CH2),
        in_specs=[
            pl.BlockSpec((1, 1, _L), lambda b, c: (b, 0, 0)),
            pl.BlockSpec((1, _C, _CH2), lambda b, c: (b, 0, c)),
            pl.BlockSpec((_H, _C), lambda b, c: (0, 0)),
            pl.BlockSpec((1, _H), lambda b, c: (0, 0)),
        ],
        out_specs=[
            pl.BlockSpec((1, 1, _KEEP), lambda b, c: (b, 0, 0)),
            pl.BlockSpec((1, 1, _KEEP), lambda b, c: (b, 0, 0)),
            pl.BlockSpec((1, 1, _KEEP), lambda b, c: (b, 0, 0)),
            pl.BlockSpec((1, _KEEP, 128), lambda b, c: (b, 0, 0)),
            pl.BlockSpec((1, 1, _KEEP), lambda b, c: (b, 0, 0)),
        ],
        out_shape=[
            jax.ShapeDtypeStruct((_B, 1, _KEEP), jnp.float32),
            jax.ShapeDtypeStruct((_B, 1, _KEEP), jnp.int32),
            jax.ShapeDtypeStruct((_B, 1, _KEEP), jnp.int32),
            jax.ShapeDtypeStruct((_B, _KEEP, 128), jnp.float32),
            jax.ShapeDtypeStruct((_B, 1, _KEEP), jnp.int32),
        ],
        scratch_shapes=[pltpu.VMEM((_KEEP, _H), jnp.float32),
                        pltpu.VMEM((_KEEP, _C), jnp.float32),
                        pltpu.VMEM((_KEEP, 1), jnp.int32)],
    )(colmax, xb, W1b, b1r)

    return v, hh, ll, spt, qi


def _p3y_body(v_ref, hh_ref, ll_ref, w2_ref, b2_ref, y_ref,
              ot0_ref, yg0_ref, yg1_ref):
    lb = pl.program_id(1)

    @pl.when(lb == 0)
    def _():
        vv = v_ref[0]
        v0 = vv.astype(jnp.bfloat16).astype(jnp.float32)
        ot = (lax.broadcasted_iota(jnp.int32, (_H, _KEEP), 0) == hh_ref[0]
              ).astype(jnp.float32)
        ot0_ref[...] = (ot * v0).astype(jnp.bfloat16)
        wg = lax.dot_general(w2_ref[...], ot0_ref[...],
                             (((1,), (0,)), ((), ())),
                             preferred_element_type=jnp.float32)
        yg0 = wg.astype(jnp.bfloat16)
        yg0_ref[...] = yg0
        yg1_ref[...] = (wg - yg0.astype(jnp.float32)).astype(jnp.bfloat16)

    l0 = lb * _LB3
    pt = (l0 + lax.broadcasted_iota(jnp.int32, (_LB3, _KEEP), 0) == ll_ref[0]
          ).astype(jnp.bfloat16)
    y_ref[0] = (
        lax.dot_general(yg0_ref[...], pt, (((1,), (1,)), ((), ())),
                        preferred_element_type=jnp.float32)
        + lax.dot_general(yg1_ref[...], pt, (((1,), (1,)), ((), ())),
                          preferred_element_type=jnp.float32)
        + b2_ref[...])


def kernel(x, W1, b1, W2, b2):
    W2b = W2.astype(jnp.bfloat16)
    b2c = b2.reshape(_C, 1)

    fill = pl.kernel(
        _fill_body,
        out_type=jax.ShapeDtypeStruct((_B, _H, _L), jnp.float32),
        mesh=_sc_mesh(),
        scratch_types=[pltpu.VMEM((_ZR, _L), jnp.float32)],
    )
    spz = fill()

    v, hh, ll, spt, qi = _tc_front(x, W1, b1, W2, b2)

    y = pl.pallas_call(
        _p3y_body,
        grid=(_B, _L // _LB3),
        in_specs=[
            pl.BlockSpec((1, 1, _KEEP), lambda b, l: (b, 0, 0)),
            pl.BlockSpec((1, 1, _KEEP), lambda b, l: (b, 0, 0)),
            pl.BlockSpec((1, 1, _KEEP), lambda b, l: (b, 0, 0)),
            pl.BlockSpec((_C, _H), lambda b, l: (0, 0)),
            pl.BlockSpec((_C, 1), lambda b, l: (0, 0)),
        ],
        out_specs=pl.BlockSpec((1, _C, _LB3), lambda b, l: (b, 0, l)),
        out_shape=jax.ShapeDtypeStruct((_B, _C, _L), jnp.float32),
        scratch_shapes=[pltpu.VMEM((_H, _KEEP), jnp.bfloat16),
                        pltpu.VMEM((_C, _KEEP), jnp.bfloat16),
                        pltpu.VMEM((_C, _KEEP), jnp.bfloat16)],
    )(v, hh, ll, W2b, b2c)

    sptf = spt.reshape(_B * _KEEP, 128)
    qif = qi.reshape(_B * _KEEP)
    sp_ref = jax.new_ref(spz.reshape(_B * _H * (_L // 128), 128))
    scat = pl.kernel(
        _scat_body,
        out_type=(),
        mesh=_sc_mesh(),
        scratch_types=[pltpu.VMEM((8, 128), jnp.float32),
                       pltpu.VMEM((8,), jnp.int32),
                       pltpu.SemaphoreType.DMA],
    )
    scat(sptf, qif, sp_ref)
    return (y, sp_ref[...].reshape(_B, _H, _L))


# speedup vs baseline: 1.9851x; 1.9851x over previous
import jax
import jax.numpy as jnp
from jax import lax
from jax.experimental import pallas as pl
from jax.experimental.pallas import tpu as pltpu
from jax.experimental.pallas import tpu_sc as plsc

_B, _C, _L = 4, 768, 8192
_H = 2048
_KEEP = 64
_LB1 = 1024
_CH2 = 2048
_LB3 = 1024
_NEG = -3.0e38
_PREC = lax.Precision.HIGHEST
_ZR = 8


def _p1_body(x_ref, w1_ref, b1_ref, out_ref):
    hb = lax.dot_general(w1_ref[...], x_ref[0], (((1,), (0,)), ((), ())),
                         preferred_element_type=jnp.float32)
    hb = hb + b1_ref[...]
    out_ref[0, 0, :] = jnp.max(hb, axis=0)


def _p2_body(cm_ref, x_ref, w1_ref, b1_ref,
             v_ref, hh_ref, ll_ref, spt_ref, hq_ref, lq_ref,
             hgT_ref, xgT_ref, cols_ref):
    iota_l = lax.broadcasted_iota(jnp.int32, (1, _L), 1)
    iota_kr = lax.broadcasted_iota(jnp.int32, (1, _KEEP), 1)
    iota_kc = lax.broadcasted_iota(jnp.int32, (_KEEP, 1), 0)
    iota_h = lax.broadcasted_iota(jnp.int32, (1, _H), 1)
    c = pl.program_id(1)

    @pl.when(c == 0)
    def _():
        def sel_body(k, carry):
            cm, cols = carry
            m = jnp.max(cm)
            j = jnp.min(jnp.where(cm == m, iota_l, _L))
            cm = jnp.where(iota_l == j, _NEG, cm)
            cols = jnp.where(iota_kc == k, j, cols)
            return cm, cols

        cm0 = cm_ref[0]
        cols0 = jnp.zeros((_KEEP, 1), jnp.int32)
        _, cols = lax.fori_loop(0, _KEEP, sel_body, (cm0, cols0))
        cols_ref[...] = cols
        xgT_ref[...] = jnp.zeros((_KEEP, _C), jnp.float32)

    cols = cols_ref[...]
    selm = (c * _CH2 + lax.broadcasted_iota(jnp.int32, (_KEEP, _CH2), 1)
            == cols).astype(jnp.bfloat16)
    xgT_ref[...] += lax.dot_general(selm, x_ref[0], (((1,), (1,)), ((), ())),
                                    preferred_element_type=jnp.float32)

    @pl.when(c == _L // _CH2 - 1)
    def _():
        hgT = lax.dot_general(xgT_ref[...].astype(jnp.bfloat16), w1_ref[...],
                              (((1,), (1,)), ((), ())),
                              preferred_element_type=jnp.float32)
        hgT = hgT + b1_ref[...]
        hgT_ref[...] = hgT
        rmax0 = jnp.max(hgT, axis=1, keepdims=True)

        iota_qc = lax.broadcasted_iota(jnp.int32, (8 * _KEEP, 1), 0)

        def ext_body(k, carry):
            rmax, v, hh, ll, llc, hd8x, ld128x, hm8x, vx = carry
            m = jnp.max(rmax)
            j = jnp.min(jnp.where(rmax == m, iota_kc, _KEEP))
            row = hgT_ref[pl.ds(j, 1), :]
            i = jnp.min(jnp.where(row == m, iota_h, _H))
            masked = jnp.where(iota_h == i, _NEG, row)
            hgT_ref[pl.ds(j, 1), :] = masked
            nm = jnp.max(masked)
            rmax = jnp.where(iota_kc == j, nm, rmax)
            lcol = jnp.min(jnp.where(iota_kc == j, cols, _L))
            v = jnp.where(iota_kr == k, m, v)
            hh = jnp.where(iota_kr == k, i, hh)
            ll = jnp.where(iota_kr == k, lcol, ll)
            llc = jnp.where(iota_kc == k, lcol, llc)
            qsel = (iota_qc // 8) == k
            hd8x = jnp.where(qsel, i // 8, hd8x)
            ld128x = jnp.where(qsel, lcol // 128, ld128x)
            hm8x = jnp.where(qsel, i % 8, hm8x)
            vx = jnp.where(qsel, m, vx)
            return rmax, v, hh, ll, llc, hd8x, ld128x, hm8x, vx

        z0r = jnp.zeros((1, _KEEP), jnp.int32)
        v0r = jnp.zeros((1, _KEEP), jnp.float32)
        z0c = jnp.zeros((_KEEP, 1), jnp.int32)
        z0q = jnp.zeros((8 * _KEEP, 1), jnp.int32)
        v0q = jnp.zeros((8 * _KEEP, 1), jnp.float32)
        _, v, hh, ll, llc, hd8x, ld128x, hm8x, vx = lax.fori_loop(
            0, _KEEP, ext_body,
            (rmax0, v0r, z0r, z0r, z0c, z0q, z0q, z0q, v0q))

        hd8r = hh // 8
        ld128r = ll // 128
        hm8r = hh % 8
        same_tile = (hd8x == hd8r) & (ld128x == ld128r)
        wmat = (same_tile & ((iota_qc % 8) == hm8r)).astype(jnp.float32) * v
        p128 = (lax.broadcasted_iota(jnp.int32, (_KEEP, 128), 1)
                == (llc % 128)).astype(jnp.float32)
        tiles = lax.dot_general(wmat, p128, (((1,), (0,)), ((), ())),
                                preferred_element_type=jnp.float32,
                                precision=_PREC)
        v_ref[0] = v
        hh_ref[0] = hh
        ll_ref[0] = ll
        spt_ref[0] = tiles
        hq_ref[0] = hd8r * 8
        lq_ref[0] = ld128r * 128


def _sc_mesh():
    return plsc.VectorSubcoreMesh(core_axis_name="c", subcore_axis_name="s")


def _fill_body(spz_ref, zbuf):
    wid = lax.axis_index("s") * 2 + lax.axis_index("c")
    z16 = jnp.zeros((16,), jnp.float32)

    def zb(t, carry):
        r = t // (_L // 16)
        i = t % (_L // 16)
        zbuf[r, pl.ds(i * 16, 16)] = z16
        return carry

    lax.fori_loop(0, _ZR * (_L // 16), zb, 0)
    b = wid // 8
    sub = wid % 8
    for j in range(256 // _ZR):
        h0 = sub * 256 + j * _ZR
        pltpu.sync_copy(zbuf, spz_ref.at[b, pl.ds(h0, _ZR), :])


def _scat_body(spt_hbm, hq_hbm, lq_hbm, spz_ref, tbuf, hv, lv):
    wid = lax.axis_index("s") * 2 + lax.axis_index("c")
    pltpu.sync_copy(hq_hbm, hv)
    pltpu.sync_copy(lq_hbm, lv)
    iota16 = lax.broadcasted_iota(jnp.int32, (16,), 0)
    for j in range(8):
        p = wid * 8 + j
        b = p // _KEEP
        lane = p % 16
        hvec = hv[pl.ds((p // 16) * 16, 16)]
        lvec = lv[pl.ds((p // 16) * 16, 16)]
        h0 = lax.reduce_max(jnp.where(iota16 == lane, hvec, -1), (0,))
        l0 = lax.reduce_max(jnp.where(iota16 == lane, lvec, -1), (0,))
        h0 = pl.multiple_of(h0, 8)
        l0 = pl.multiple_of(l0, 128)
        pltpu.sync_copy(spt_hbm.at[pl.ds(pl.multiple_of(p * 8, 8), 8)], tbuf)
        pltpu.sync_copy(tbuf, spz_ref.at[b, pl.ds(h0, 8), pl.ds(l0, 128)])


def _tc_front(x, W1, b1, W2, b2):
    xb = x.astype(jnp.bfloat16)
    W1b = W1.astype(jnp.bfloat16)
    W2b = W2.astype(jnp.bfloat16)
    b1c = b1.reshape(_H, 1)
    b1r = b1.reshape(1, _H)
    b2r = b2.reshape(1, _C)

    colmax = pl.pallas_call(
        _p1_body,
        grid=(_B, _L // _LB1),
        in_specs=[
            pl.BlockSpec((1, _C, _LB1), lambda b, l: (b, 0, l)),
            pl.BlockSpec((_H, _C), lambda b, l: (0, 0)),
            pl.BlockSpec((_H, 1), lambda b, l: (0, 0)),
        ],
        out_specs=pl.BlockSpec((1, 1, _LB1), lambda b, l: (b, 0, l)),
        out_shape=jax.ShapeDtypeStruct((_B, 1, _L), jnp.float32),
    )(xb, W1b, b1c)

    v, hh, ll, spt, hq, lq = pl.pallas_call(
        _p2_body,
        grid=(_B, _L // _CH2),
        in_specs=[
            pl.BlockSpec((1, 1, _L), lambda b, c: (b, 0, 0)),
            pl.BlockSpec((1, _C, _CH2), lambda b, c: (b, 0, c)),
            pl.BlockSpec((_H, _C), lambda b, c: (0, 0)),
            pl.BlockSpec((1, _H), lambda b, c: (0, 0)),
        ],
        out_specs=[
            pl.BlockSpec((1, 1, _KEEP), lambda b, c: (b, 0, 0)),
            pl.BlockSpec((1, 1, _KEEP), lambda b, c: (b, 0, 0)),
            pl.BlockSpec((1, 1, _KEEP), lambda b, c: (b, 0, 0)),
            pl.BlockSpec((1, 8 * _KEEP, 128), lambda b, c: (b, 0, 0)),
            pl.BlockSpec((1, 1, _KEEP), lambda b, c: (b, 0, 0)),
            pl.BlockSpec((1, 1, _KEEP), lambda b, c: (b, 0, 0)),
        ],
        out_shape=[
            jax.ShapeDtypeStruct((_B, 1, _KEEP), jnp.float32),
            jax.ShapeDtypeStruct((_B, 1, _KEEP), jnp.int32),
            jax.ShapeDtypeStruct((_B, 1, _KEEP), jnp.int32),
            jax.ShapeDtypeStruct((_B, 8 * _KEEP, 128), jnp.float32),
            jax.ShapeDtypeStruct((_B, 1, _KEEP), jnp.int32),
            jax.ShapeDtypeStruct((_B, 1, _KEEP), jnp.int32),
        ],
        scratch_shapes=[pltpu.VMEM((_KEEP, _H), jnp.float32),
                        pltpu.VMEM((_KEEP, _C), jnp.float32),
                        pltpu.VMEM((_KEEP, 1), jnp.int32)],
    )(colmax, xb, W1b, b1r)

    return v, hh, ll, spt, hq, lq


def _p3y_body(v_ref, hh_ref, ll_ref, w2_ref, b2_ref, y_ref,
              ot0_ref, yg0_ref, yg1_ref):
    lb = pl.program_id(1)

    @pl.when(lb == 0)
    def _():
        vv = v_ref[0]
        v0 = vv.astype(jnp.bfloat16).astype(jnp.float32)
        ot = (lax.broadcasted_iota(jnp.int32, (_H, _KEEP), 0) == hh_ref[0]
              ).astype(jnp.float32)
        ot0_ref[...] = (ot * v0).astype(jnp.bfloat16)
        wg = lax.dot_general(w2_ref[...], ot0_ref[...],
                             (((1,), (0,)), ((), ())),
                             preferred_element_type=jnp.float32)
        yg0 = wg.astype(jnp.bfloat16)
        yg0_ref[...] = yg0
        yg1_ref[...] = (wg - yg0.astype(jnp.float32)).astype(jnp.bfloat16)

    l0 = lb * _LB3
    pt = (l0 + lax.broadcasted_iota(jnp.int32, (_LB3, _KEEP), 0) == ll_ref[0]
          ).astype(jnp.bfloat16)
    y_ref[0] = (
        lax.dot_general(yg0_ref[...], pt, (((1,), (1,)), ((), ())),
                        preferred_element_type=jnp.float32)
        + lax.dot_general(yg1_ref[...], pt, (((1,), (1,)), ((), ())),
                          preferred_element_type=jnp.float32)
        + b2_ref[...])


def kernel(x, W1, b1, W2, b2):
    W2b = W2.astype(jnp.bfloat16)
    b2c = b2.reshape(_C, 1)

    fill = pl.kernel(
        _fill_body,
        out_type=jax.ShapeDtypeStruct((_B, _H, _L), jnp.float32),
        mesh=_sc_mesh(),
        scratch_types=[pltpu.VMEM((_ZR, _L), jnp.float32)],
    )
    spz = fill()

    v, hh, ll, spt, hq, lq = _tc_front(x, W1, b1, W2, b2)

    y = pl.pallas_call(
        _p3y_body,
        grid=(_B, _L // _LB3),
        in_specs=[
            pl.BlockSpec((1, 1, _KEEP), lambda b, l: (b, 0, 0)),
            pl.BlockSpec((1, 1, _KEEP), lambda b, l: (b, 0, 0)),
            pl.BlockSpec((1, 1, _KEEP), lambda b, l: (b, 0, 0)),
            pl.BlockSpec((_C, _H), lambda b, l: (0, 0)),
            pl.BlockSpec((_C, 1), lambda b, l: (0, 0)),
        ],
        out_specs=pl.BlockSpec((1, _C, _LB3), lambda b, l: (b, 0, l)),
        out_shape=jax.ShapeDtypeStruct((_B, _C, _L), jnp.float32),
        scratch_shapes=[pltpu.VMEM((_H, _KEEP), jnp.bfloat16),
                        pltpu.VMEM((_C, _KEEP), jnp.bfloat16),
                        pltpu.VMEM((_C, _KEEP), jnp.bfloat16)],
    )(v, hh, ll, W2b, b2c)

    sptf = spt.reshape(_B * 8 * _KEEP, 128)
    hqf = hq.reshape(_B * _KEEP)
    lqf = lq.reshape(_B * _KEEP)
    sp_ref = jax.new_ref(spz)
    scat = pl.kernel(
        _scat_body,
        out_type=(),
        mesh=_sc_mesh(),
        scratch_types=[pltpu.VMEM((8, 128), jnp.float32),
                       pltpu.VMEM((_B * _KEEP,), jnp.int32),
                       pltpu.VMEM((_B * _KEEP,), jnp.int32)],
        compiler_params=pltpu.CompilerParams(needs_layout_passes=False),
    )
    scat(sptf, hqf, lqf, sp_ref)
    return (y, sp_ref[...])
